# baseline (speedup 1.0000x reference)
import functools

import jax
import jax.numpy as jnp
from jax import lax
from jax.experimental import pallas as pl
from jax.experimental.pallas import tpu as pltpu
from jax.experimental.pallas import tpu_sc as plsc

VOCAB = 1000000
EMB = 16
BATCH = 16384
GROUP = 8
NGROUPS = VOCAB // GROUP

NC = 2
NS = 16
NW = NC * NS
BPW = BATCH // NW
ICHUNK = 128
NCK = BPW // ICHUNK


def _rsqrt_nr(x):
    i = lax.bitcast_convert_type(x, jnp.int32)
    i = jnp.int32(0x5F3759DF) - lax.shift_right_logical(i, 1)
    y = lax.bitcast_convert_type(i, jnp.float32)
    half = jnp.float32(0.5) * x
    for _ in range(3):
        y = y * (jnp.float32(1.5) - half * y * y)
    return y


def _make_sc_kernel():
    mesh = plsc.VectorSubcoreMesh(core_axis_name="c", subcore_axis_name="s")

    @functools.partial(
        pl.kernel,
        mesh=mesh,
        out_type=jax.ShapeDtypeStruct((BATCH,), jnp.float32),
        compiler_params=pltpu.CompilerParams(
            needs_layout_passes=False, use_tc_tiling_on_sc=False),
        scratch_types=[
            pltpu.VMEM((2 * BPW,), jnp.int32),
            pltpu.VMEM((BPW,), jnp.int32),
            pltpu.VMEM((BPW,), jnp.int32),
            pltpu.VMEM((BPW, EMB), jnp.float32),
            pltpu.VMEM((BPW, EMB), jnp.float32),
            pltpu.VMEM((BPW,), jnp.float32),
            pltpu.SemaphoreType.DMA,
        ],
    )
    def sc_kernel(inp_hbm, tab_hbm, out_hbm, iv, ia, ib, ar, br, outv, sem):
        wid = lax.axis_index("s") * NC + lax.axis_index("c")
        base = wid * BPW

        pltpu.sync_copy(inp_hbm.at[pl.ds(2 * base, 2 * BPW)], iv)

        lane = lax.iota(jnp.int32, 16)

        def prep(k, _):
            pos = 2 * (k * 16 + lane)
            ia[pl.ds(k * 16, 16)] = plsc.load_gather(iv, [pos])
            ib[pl.ds(k * 16, 16)] = plsc.load_gather(iv, [pos + 1])
            return 0

        lax.fori_loop(0, BPW // 16, prep, 0)

        copies = []
        for c in range(NCK):
            sl = pl.ds(c * ICHUNK, ICHUNK)
            copies.append(pltpu.async_copy(
                tab_hbm.at[ia.at[sl]], ar.at[sl], sem))
            copies.append(pltpu.async_copy(
                tab_hbm.at[ib.at[sl]], br.at[sl], sem))
        for cp in copies:
            cp.wait()

        eps2 = jnp.full((16,), 1e-16, jnp.float32)

        def body(g, _):
            rows = g * 16 + lane
            dot = jnp.zeros((16,), jnp.float32)
            a2 = jnp.zeros((16,), jnp.float32)
            b2 = jnp.zeros((16,), jnp.float32)
            for d in range(EMB):
                cols = jnp.full((16,), d, jnp.int32)
                av = plsc.load_gather(ar, [rows, cols])
                bv = plsc.load_gather(br, [rows, cols])
                dot = dot + av * bv
                a2 = a2 + av * av
                b2 = b2 + bv * bv
            denom2 = jnp.maximum(a2 * b2, eps2)
            outv[pl.ds(g * 16, 16)] = dot * _rsqrt_nr(denom2)
            return 0

        lax.fori_loop(0, BPW // 16, body, 0)

        pltpu.sync_copy(outv, out_hbm.at[pl.ds(base, BPW)])

    return sc_kernel


_sc_kernel = _make_sc_kernel()


def kernel(input, table):
    inp = input.reshape(2 * BATCH)
    tab_fm = lax.optimization_barrier(table.T)
    tab = (tab_fm.reshape(EMB, NGROUPS, GROUP)
           .transpose(1, 2, 0)
           .reshape(VOCAB, EMB))
    return _sc_kernel(inp, tab)


# speedup vs baseline: 1.5950x; 1.5950x over previous
import functools

import jax
import jax.numpy as jnp
from jax import lax
from jax.experimental import pallas as pl
from jax.experimental.pallas import tpu as pltpu
from jax.experimental.pallas import tpu_sc as plsc

VOCAB = 1000000
EMB = 16
BATCH = 16384
GROUP = 128 // EMB
NGROUPS = VOCAB // GROUP

NC = 2
NS = 16
NW = NC * NS
BPW = BATCH // NW
CHUNK = 128
NCK = BPW // CHUNK


def _rsqrt_nr(x):
    i = lax.bitcast_convert_type(x, jnp.int32)
    i = jnp.int32(0x5F3759DF) - lax.shift_right_logical(i, 1)
    y = lax.bitcast_convert_type(i, jnp.float32)
    half = jnp.float32(0.5) * x
    for _ in range(3):
        y = y * (jnp.float32(1.5) - half * y * y)
    return y


def _make_sc_kernel():
    mesh = plsc.VectorSubcoreMesh(core_axis_name="c", subcore_axis_name="s")

    @functools.partial(
        pl.kernel,
        mesh=mesh,
        out_type=jax.ShapeDtypeStruct((BATCH,), jnp.float32),
        compiler_params=pltpu.CompilerParams(needs_layout_passes=False),
        scratch_types=[
            pltpu.VMEM((2 * BPW,), jnp.int32),
            pltpu.VMEM((BPW,), jnp.int32),
            pltpu.VMEM((BPW,), jnp.int32),
            pltpu.VMEM((BPW,), jnp.int32),
            pltpu.VMEM((BPW,), jnp.int32),
            pltpu.VMEM((CHUNK, 128), jnp.float32),
            pltpu.VMEM((CHUNK, 128), jnp.float32),
            pltpu.VMEM((BPW,), jnp.float32),
            pltpu.SemaphoreType.DMA,
        ],
    )
    def sc_kernel(inp_hbm, table_hbm, out_hbm,
                  iv, ja, jb, ra, rb, ag, bg, outv, sem):
        wid = lax.axis_index("s") * NC + lax.axis_index("c")
        base = wid * BPW

        pltpu.sync_copy(inp_hbm.at[pl.ds(2 * base, 2 * BPW)], iv)

        lane = lax.iota(jnp.int32, 16)

        def prep(k, _):
            pos = 2 * (k * 16 + lane)
            for off, jref, rref in ((0, ja, ra), (1, jb, rb)):
                idx = plsc.load_gather(iv, [pos + off])
                jref[pl.ds(k * 16, 16)] = lax.shift_right_logical(idx, 3)
                rref[pl.ds(k * 16, 16)] = (idx & (GROUP - 1)) * EMB
            return 0

        lax.fori_loop(0, BPW // 16, prep, 0)

        eps2 = jnp.full((16,), 1e-16, jnp.float32)

        for c in range(NCK):
            cpa = pltpu.async_copy(
                table_hbm.at[ja.at[pl.ds(c * CHUNK, CHUNK)]], ag, sem)
            cpb = pltpu.async_copy(
                table_hbm.at[jb.at[pl.ds(c * CHUNK, CHUNK)]], bg, sem)
            cpa.wait()
            cpb.wait()

            def cbody(g, _, c=c):
                rows = g * 16 + lane
                pbase = c * CHUNK + g * 16
                ca = plsc.load_gather(ra, [pbase + lane])
                cb = plsc.load_gather(rb, [pbase + lane])
                dot = jnp.zeros((16,), jnp.float32)
                a2 = jnp.zeros((16,), jnp.float32)
                b2 = jnp.zeros((16,), jnp.float32)
                for d in range(EMB):
                    av = plsc.load_gather(ag, [rows, ca + d])
                    bv = plsc.load_gather(bg, [rows, cb + d])
                    dot = dot + av * bv
                    a2 = a2 + av * av
                    b2 = b2 + bv * bv
                denom2 = jnp.maximum(a2 * b2, eps2)
                outv[pl.ds(pbase, 16)] = dot * _rsqrt_nr(denom2)
                return 0

            lax.fori_loop(0, CHUNK // 16, cbody, 0)

        pltpu.sync_copy(outv, out_hbm.at[pl.ds(base, BPW)])

    return sc_kernel


_sc_kernel = _make_sc_kernel()


def kernel(input, table):
    inp = input.reshape(2 * BATCH)
    tab_fm = lax.optimization_barrier(table.T)
    tab_t = lax.optimization_barrier(
        tab_fm.reshape(EMB, NGROUPS, GROUP).transpose(1, 2, 0))
    tab = tab_t.reshape(NGROUPS, GROUP * EMB)
    return _sc_kernel(inp, tab)


# speedup vs baseline: 1.6089x; 1.0087x over previous
import functools

import jax
import jax.numpy as jnp
from jax import lax
from jax.experimental import pallas as pl
from jax.experimental.pallas import tpu as pltpu
from jax.experimental.pallas import tpu_sc as plsc

VOCAB = 1000000
EMB = 16
BATCH = 16384
GROUP = 128 // EMB
NGROUPS = VOCAB // GROUP

NC = 2
NS = 16
NW = NC * NS
BPW = BATCH // NW
CHUNK = 128
NCK = BPW // CHUNK


def _rsqrt_nr(x):
    i = lax.bitcast_convert_type(x, jnp.int32)
    i = jnp.int32(0x5F3759DF) - lax.shift_right_logical(i, 1)
    y = lax.bitcast_convert_type(i, jnp.float32)
    half = jnp.float32(0.5) * x
    for _ in range(3):
        y = y * (jnp.float32(1.5) - half * y * y)
    return y


def _make_sc_kernel():
    mesh = plsc.VectorSubcoreMesh(core_axis_name="c", subcore_axis_name="s")

    @functools.partial(
        pl.kernel,
        mesh=mesh,
        out_type=jax.ShapeDtypeStruct((BATCH,), jnp.float32),
        compiler_params=pltpu.CompilerParams(needs_layout_passes=False),
        scratch_types=[
            pltpu.VMEM((2 * BPW,), jnp.int32),
            pltpu.VMEM((BPW,), jnp.int32),
            pltpu.VMEM((BPW,), jnp.int32),
            pltpu.VMEM((BPW,), jnp.int32),
            pltpu.VMEM((BPW,), jnp.int32),
            pltpu.VMEM((CHUNK, 128), jnp.float32),
            pltpu.VMEM((CHUNK, 128), jnp.float32),
            pltpu.VMEM((BPW,), jnp.float32),
            pltpu.SemaphoreType.DMA,
        ],
    )
    def sc_kernel(inp_hbm, table_hbm, out_hbm,
                  iv, ja, jb, ra, rb, ag, bg, outv, sem):
        wid = lax.axis_index("s") * NC + lax.axis_index("c")
        base = wid * BPW

        pltpu.sync_copy(inp_hbm.at[pl.ds(2 * base, 2 * BPW)], iv)

        lane = lax.iota(jnp.int32, 16)

        def prep(k, _):
            pos = 2 * (k * 16 + lane)
            for off, jref, rref in ((0, ja, ra), (1, jb, rb)):
                idx = plsc.load_gather(iv, [pos + off])
                jref[pl.ds(k * 16, 16)] = lax.shift_right_logical(idx, 3)
                rref[pl.ds(k * 16, 16)] = idx & (GROUP - 1)
            return 0

        lax.fori_loop(0, BPW // 16, prep, 0)

        eps2 = jnp.full((16,), 1e-16, jnp.float32)

        for c in range(NCK):
            cpa = pltpu.async_copy(
                table_hbm.at[ja.at[pl.ds(c * CHUNK, CHUNK)]], ag, sem)
            cpb = pltpu.async_copy(
                table_hbm.at[jb.at[pl.ds(c * CHUNK, CHUNK)]], bg, sem)
            cpa.wait()
            cpb.wait()

            def cbody(g, _, c=c):
                rows = g * 16 + lane
                pbase = c * CHUNK + g * 16
                ca = plsc.load_gather(ra, [pbase + lane])
                cb = plsc.load_gather(rb, [pbase + lane])
                dot = jnp.zeros((16,), jnp.float32)
                a2 = jnp.zeros((16,), jnp.float32)
                b2 = jnp.zeros((16,), jnp.float32)
                for d in range(EMB):
                    av = plsc.load_gather(ag, [rows, ca + d * GROUP])
                    bv = plsc.load_gather(bg, [rows, cb + d * GROUP])
                    dot = dot + av * bv
                    a2 = a2 + av * av
                    b2 = b2 + bv * bv
                denom2 = jnp.maximum(a2 * b2, eps2)
                outv[pl.ds(pbase, 16)] = dot * _rsqrt_nr(denom2)
                return 0

            lax.fori_loop(0, CHUNK // 16, cbody, 0)

        pltpu.sync_copy(outv, out_hbm.at[pl.ds(base, BPW)])

    return sc_kernel


_sc_kernel = _make_sc_kernel()


def kernel(input, table):
    inp = input.reshape(2 * BATCH)
    tab_fm = lax.optimization_barrier(table.T)
    tab = (tab_fm.reshape(EMB, NGROUPS, GROUP)
           .transpose(1, 0, 2)
           .reshape(NGROUPS, GROUP * EMB))
    return _sc_kernel(inp, tab)
